# baseline (speedup 1.0000x reference)
import functools

import jax
import jax.numpy as jnp
from jax import lax
from jax.experimental import pallas as pl
from jax.experimental.pallas import tpu as pltpu
from jax.experimental.pallas import tpu_sc as plsc

BATCH = 16384
EMBED_DIM = 64
NUM_ROWS = 1000000

_info = plsc.get_sparse_core_info()
_NC, _NS = _info.num_cores, _info.num_subcores
_NW = _NC * _NS
_B_PER_W = BATCH // _NW
_LANES = 16
_NBUF = 10

_mesh = plsc.VectorSubcoreMesh(core_axis_name="c", subcore_axis_name="s")


@functools.partial(
    pl.kernel,
    mesh=_mesh,
    compiler_params=pltpu.CompilerParams(needs_layout_passes=False),
    out_type=(
        jax.ShapeDtypeStruct((BATCH * EMBED_DIM,), jnp.float32),
        jax.ShapeDtypeStruct((BATCH * EMBED_DIM,), jnp.float32),
    ),
    scratch_types=[
        pltpu.VMEM((_B_PER_W,), jnp.int32),
        pltpu.VMEM((_B_PER_W,), jnp.int32),
        pltpu.VMEM((_B_PER_W,), jnp.int32),
        pltpu.VMEM((_B_PER_W,), jnp.int32),
        pltpu.VMEM((16,), jnp.int32),
        pltpu.VMEM((EMBED_DIM, 2 * EMBED_DIM), jnp.float32),
        pltpu.VMEM((EMBED_DIM, 2 * EMBED_DIM), jnp.float32),
        pltpu.VMEM((EMBED_DIM, 2 * EMBED_DIM), jnp.float32),
        pltpu.VMEM((EMBED_DIM, 2 * EMBED_DIM), jnp.float32),
        pltpu.VMEM((EMBED_DIM, 2 * EMBED_DIM), jnp.float32),
        pltpu.VMEM((EMBED_DIM, 2 * EMBED_DIM), jnp.float32),
        pltpu.VMEM((EMBED_DIM, 2 * EMBED_DIM), jnp.float32),
        pltpu.VMEM((EMBED_DIM, 2 * EMBED_DIM), jnp.float32),
        pltpu.VMEM((EMBED_DIM, 2 * EMBED_DIM), jnp.float32),
        pltpu.VMEM((EMBED_DIM, 2 * EMBED_DIM), jnp.float32),
        pltpu.VMEM((_B_PER_W * EMBED_DIM,), jnp.float32),
        pltpu.SemaphoreType.DMA,
        pltpu.SemaphoreType.DMA,
        pltpu.SemaphoreType.DMA,
        pltpu.SemaphoreType.DMA,
        pltpu.SemaphoreType.DMA,
        pltpu.SemaphoreType.DMA,
        pltpu.SemaphoreType.DMA,
        pltpu.SemaphoreType.DMA,
        pltpu.SemaphoreType.DMA,
        pltpu.SemaphoreType.DMA,
        pltpu.SemaphoreType.DMA,
    ],
)
def _gather_sorted(sids_u, dense_u, estart_u, korig_u, counts_u,
                   sids_i, dense_i, estart_i, korig_i, counts_i,
                   user_t, item_t, u_inter, i_inter,
                   sids_v, dense_v, est_v, k_v, cnt_v,
                   b0, b1, b2, b3, b4, b5, b6, b7, b8, b9, gbuf,
                   s0, s1, s2, s3, s4, s5, s6, s7, s8, s9, wsem):
    wid = lax.axis_index("s") * _NC + lax.axis_index("c")
    bufs = (b0, b1, b2, b3, b4, b5, b6, b7, b8, b9)
    sems = (s0, s1, s2, s3, s4, s5, s6, s7, s8, s9)
    lane = lax.iota(jnp.int32, _LANES)

    def sread(ref, i):
        v = ref[pl.ds(pl.multiple_of((i >> 4) << 4, 16), _LANES)]
        return jnp.max(jnp.where(lane == (i & 15), v, 0))

    def do_table(sids, dense, estart, korig, counts, tab, inter):
        pltpu.sync_copy(sids.at[pl.ds(wid * _B_PER_W, _B_PER_W)], sids_v)
        pltpu.sync_copy(dense.at[pl.ds(wid * _B_PER_W, _B_PER_W)], dense_v)
        pltpu.sync_copy(estart.at[pl.ds(wid * _B_PER_W, _B_PER_W)], est_v)
        pltpu.sync_copy(korig.at[pl.ds(wid * _B_PER_W, _B_PER_W)], k_v)
        pltpu.sync_copy(counts.at[pl.ds(wid * 16, 16)], cnt_v)
        n = sread(cnt_v, 0)

        def fetch(j, b):
            @pl.when(j < n)
            def _():
                off = pl.multiple_of(sread(dense_v, j) * 128, 128)
                pltpu.async_copy(tab.at[:, pl.ds(off, 2 * EMBED_DIM)],
                                 bufs[b], sems[b])

        for b in range(_NBUF):
            fetch(b, b)

        def group(g, _):
            for b in range(_NBUF):
                j = g * _NBUF + b

                @pl.when(j < n)
                def _(b=b):
                    pltpu.make_async_copy(
                        tab.at[:, pl.ds(0, 2 * EMBED_DIM)], bufs[b],
                        sems[b]).wait()

                def extract(e, _, b=b):
                    eidx = jax.lax.broadcast(e, (_LANES,))
                    col = plsc.load_gather(sids_v, [eidx]) & 127
                    for q in range(4):
                        vals = plsc.load_gather(
                            bufs[b], [lane + q * _LANES, col])
                        gbuf[pl.ds(e * EMBED_DIM + q * _LANES, _LANES)] = vals
                    pltpu.async_copy(
                        gbuf.at[pl.ds(e * EMBED_DIM, EMBED_DIM)],
                        inter.at[pl.ds(sread(k_v, e) * EMBED_DIM,
                                       EMBED_DIM)],
                        wsem)
                    return 0

                es = sread(est_v, j)
                ee = jnp.where(
                    j + 1 >= _B_PER_W, _B_PER_W,
                    sread(est_v, jnp.minimum(j + 1, _B_PER_W - 1)))
                lax.fori_loop(es, ee, extract, 0)
                fetch(j + _NBUF, b)
            return 0

        lax.fori_loop(0, (n + _NBUF - 1) // _NBUF, group, 0)
        pltpu.make_async_copy(
            inter.at[pl.ds(wid * _B_PER_W * EMBED_DIM,
                           _B_PER_W * EMBED_DIM)],
            gbuf, wsem).wait()

    do_table(sids_u, dense_u, estart_u, korig_u, counts_u, user_t, u_inter)
    do_table(sids_i, dense_i, estart_i, korig_i, counts_i, item_t, i_inter)


@functools.partial(
    pl.kernel,
    mesh=_mesh,
    compiler_params=pltpu.CompilerParams(
        use_tc_tiling_on_sc=False, needs_layout_passes=False),
    out_type=(
        jax.ShapeDtypeStruct((EMBED_DIM, BATCH), jnp.float32),
        jax.ShapeDtypeStruct((EMBED_DIM, BATCH), jnp.float32),
    ),
    scratch_types=[
        pltpu.VMEM((_B_PER_W * EMBED_DIM,), jnp.float32),
        pltpu.VMEM((EMBED_DIM, _B_PER_W), jnp.float32),
    ],
)
def _transpose_out(u_inter, i_inter, u_out, i_out, gbuf, stage):
    wid = lax.axis_index("s") * _NC + lax.axis_index("c")
    base = wid * _B_PER_W
    lane = lax.iota(jnp.int32, _LANES)

    def do_table(inter, out):
        pltpu.sync_copy(
            inter.at[pl.ds(base * EMBED_DIM, _B_PER_W * EMBED_DIM)], gbuf)

        def transpose_entry(e, _):
            ecol = jax.lax.broadcast(e, (_LANES,))
            for q in range(EMBED_DIM // _LANES):
                vals = gbuf[pl.ds(e * EMBED_DIM + q * _LANES, _LANES)]
                plsc.store_scatter(stage, [lane + q * _LANES, ecol], vals)
            return 0

        lax.fori_loop(0, _B_PER_W, transpose_entry, 0)
        pltpu.sync_copy(stage, out.at[:, pl.ds(base, _B_PER_W)])

    do_table(u_inter, u_out)
    do_table(i_inter, i_out)


def _prep(ids):
    order = jnp.argsort(ids)
    sids = jnp.take(ids, order).astype(jnp.int32)
    chunk = sids >> 7
    pos = jnp.arange(BATCH, dtype=jnp.int32)
    widx = pos // _B_PER_W
    newf = ((pos % _B_PER_W) == 0) | (chunk != jnp.roll(chunk, 1))
    sent = (((1 << 13) - 1) << 10) | _B_PER_W
    ckey = (widx << 23) | jnp.where(
        newf, (chunk << 10) | (pos % _B_PER_W), sent)
    csort = jnp.sort(ckey)
    dense = (csort >> 10) & ((1 << 13) - 1)
    estart = csort & ((1 << 10) - 1)
    counts = newf.reshape(_NW, _B_PER_W).sum(axis=1).astype(jnp.int32)
    counts_pad = jnp.pad(counts[:, None], ((0, 0), (0, 15)))
    return (sids, dense, estart, order.astype(jnp.int32),
            counts_pad.reshape(-1))


def kernel(u_ids, i_ids, user_table, item_table):
    pu = _prep(u_ids)
    pi = _prep(i_ids)
    u_inter, i_inter = _gather_sorted(
        *pu, *pi, user_table.T, item_table.T)
    u_t, i_t = _transpose_out(u_inter, i_inter)
    return (u_t.T, i_t.T)


# speedup vs baseline: 1.0284x; 1.0284x over previous
import functools

import jax
import jax.numpy as jnp
from jax import lax
from jax.experimental import pallas as pl
from jax.experimental.pallas import tpu as pltpu
from jax.experimental.pallas import tpu_sc as plsc

BATCH = 16384
EMBED_DIM = 64
NUM_ROWS = 1000000

_info = plsc.get_sparse_core_info()
_NC, _NS = _info.num_cores, _info.num_subcores
_NW = _NC * _NS
_B_PER_W = BATCH // _NW
_LANES = 16
_NBUF = 8

_mesh = plsc.VectorSubcoreMesh(core_axis_name="c", subcore_axis_name="s")


@functools.partial(
    pl.kernel,
    mesh=_mesh,
    compiler_params=pltpu.CompilerParams(needs_layout_passes=False),
    out_type=(
        jax.ShapeDtypeStruct((BATCH * EMBED_DIM,), jnp.float32),
        jax.ShapeDtypeStruct((BATCH * EMBED_DIM,), jnp.float32),
    ),
    scratch_types=[
        pltpu.VMEM((_B_PER_W,), jnp.int32),
        pltpu.VMEM((_B_PER_W,), jnp.int32),
        pltpu.VMEM((_B_PER_W,), jnp.int32),
        pltpu.VMEM((_B_PER_W,), jnp.int32),
        pltpu.VMEM((16,), jnp.int32),
        pltpu.VMEM((EMBED_DIM, 2 * EMBED_DIM), jnp.float32),
        pltpu.VMEM((EMBED_DIM, 2 * EMBED_DIM), jnp.float32),
        pltpu.VMEM((EMBED_DIM, 2 * EMBED_DIM), jnp.float32),
        pltpu.VMEM((EMBED_DIM, 2 * EMBED_DIM), jnp.float32),
        pltpu.VMEM((EMBED_DIM, 2 * EMBED_DIM), jnp.float32),
        pltpu.VMEM((EMBED_DIM, 2 * EMBED_DIM), jnp.float32),
        pltpu.VMEM((EMBED_DIM, 2 * EMBED_DIM), jnp.float32),
        pltpu.VMEM((EMBED_DIM, 2 * EMBED_DIM), jnp.float32),
        pltpu.VMEM((_B_PER_W * EMBED_DIM,), jnp.float32),
        pltpu.SemaphoreType.DMA,
        pltpu.SemaphoreType.DMA,
        pltpu.SemaphoreType.DMA,
        pltpu.SemaphoreType.DMA,
        pltpu.SemaphoreType.DMA,
        pltpu.SemaphoreType.DMA,
        pltpu.SemaphoreType.DMA,
        pltpu.SemaphoreType.DMA,
        pltpu.SemaphoreType.DMA,
    ],
)
def _gather_sorted(sids_u, dense_u, estart_u, korig_u, counts_u,
                   sids_i, dense_i, estart_i, korig_i, counts_i,
                   user_t, item_t, u_inter, i_inter,
                   sids_v, dense_v, est_v, k_v, cnt_v,
                   b0, b1, b2, b3, b4, b5, b6, b7, gbuf,
                   s0, s1, s2, s3, s4, s5, s6, s7, wsem):
    wid = lax.axis_index("s") * _NC + lax.axis_index("c")
    bufs = (b0, b1, b2, b3, b4, b5, b6, b7)
    sems = (s0, s1, s2, s3, s4, s5, s6, s7)
    lane = lax.iota(jnp.int32, _LANES)

    def sread(ref, i):
        v = ref[pl.ds(pl.multiple_of((i >> 4) << 4, 16), _LANES)]
        return jnp.max(jnp.where(lane == (i & 15), v, 0))

    def do_table(sids, dense, estart, korig, counts, tab, inter):
        pltpu.sync_copy(sids.at[pl.ds(wid * _B_PER_W, _B_PER_W)], sids_v)
        pltpu.sync_copy(dense.at[pl.ds(wid * _B_PER_W, _B_PER_W)], dense_v)
        pltpu.sync_copy(estart.at[pl.ds(wid * _B_PER_W, _B_PER_W)], est_v)
        pltpu.sync_copy(korig.at[pl.ds(wid * _B_PER_W, _B_PER_W)], k_v)
        pltpu.sync_copy(counts.at[pl.ds(wid * 16, 16)], cnt_v)
        n = sread(cnt_v, 0)

        def fetch(j, b):
            @pl.when(j < n)
            def _():
                off = pl.multiple_of(sread(dense_v, j) * 128, 128)
                pltpu.async_copy(tab.at[:, pl.ds(off, 2 * EMBED_DIM)],
                                 bufs[b], sems[b])

        for b in range(_NBUF):
            fetch(b, b)

        def group(g, _):
            for b in range(_NBUF):
                j = g * _NBUF + b

                @pl.when(j < n)
                def _(b=b):
                    pltpu.make_async_copy(
                        tab.at[:, pl.ds(0, 2 * EMBED_DIM)], bufs[b],
                        sems[b]).wait()

                def extract(e, _, b=b):
                    eidx = jax.lax.broadcast(e, (_LANES,))
                    col = plsc.load_gather(sids_v, [eidx]) & 127
                    for q in range(4):
                        vals = plsc.load_gather(
                            bufs[b], [lane + q * _LANES, col])
                        gbuf[pl.ds(e * EMBED_DIM + q * _LANES, _LANES)] = vals
                    pltpu.async_copy(
                        gbuf.at[pl.ds(e * EMBED_DIM, EMBED_DIM)],
                        inter.at[pl.ds(sread(k_v, e) * EMBED_DIM,
                                       EMBED_DIM)],
                        wsem)
                    return 0

                es = sread(est_v, j)
                ee = jnp.where(
                    j + 1 >= _B_PER_W, _B_PER_W,
                    sread(est_v, jnp.minimum(j + 1, _B_PER_W - 1)))
                lax.fori_loop(es, ee, extract, 0)
                fetch(j + _NBUF, b)
            return 0

        lax.fori_loop(0, (n + _NBUF - 1) // _NBUF, group, 0)
        pltpu.make_async_copy(
            inter.at[pl.ds(wid * _B_PER_W * EMBED_DIM,
                           _B_PER_W * EMBED_DIM)],
            gbuf, wsem).wait()

    do_table(sids_u, dense_u, estart_u, korig_u, counts_u, user_t, u_inter)
    do_table(sids_i, dense_i, estart_i, korig_i, counts_i, item_t, i_inter)


@functools.partial(
    pl.kernel,
    mesh=_mesh,
    compiler_params=pltpu.CompilerParams(
        use_tc_tiling_on_sc=False, needs_layout_passes=False),
    out_type=(
        jax.ShapeDtypeStruct((EMBED_DIM, BATCH), jnp.float32),
        jax.ShapeDtypeStruct((EMBED_DIM, BATCH), jnp.float32),
    ),
    scratch_types=[
        pltpu.VMEM((_B_PER_W * EMBED_DIM,), jnp.float32),
        pltpu.VMEM((EMBED_DIM, _B_PER_W), jnp.float32),
    ],
)
def _transpose_out(u_inter, i_inter, u_out, i_out, gbuf, stage):
    wid = lax.axis_index("s") * _NC + lax.axis_index("c")
    base = wid * _B_PER_W
    lane = lax.iota(jnp.int32, _LANES)

    def do_table(inter, out):
        pltpu.sync_copy(
            inter.at[pl.ds(base * EMBED_DIM, _B_PER_W * EMBED_DIM)], gbuf)

        def transpose_entry(e, _):
            ecol = jax.lax.broadcast(e, (_LANES,))
            for q in range(EMBED_DIM // _LANES):
                vals = gbuf[pl.ds(e * EMBED_DIM + q * _LANES, _LANES)]
                plsc.store_scatter(stage, [lane + q * _LANES, ecol], vals)
            return 0

        lax.fori_loop(0, _B_PER_W, transpose_entry, 0)
        pltpu.sync_copy(stage, out.at[:, pl.ds(base, _B_PER_W)])

    do_table(u_inter, u_out)
    do_table(i_inter, i_out)


def _prep(ids):
    order = jnp.argsort(ids)
    sids = jnp.take(ids, order).astype(jnp.int32)
    chunk = sids >> 7
    pos = jnp.arange(BATCH, dtype=jnp.int32)
    widx = pos // _B_PER_W
    newf = ((pos % _B_PER_W) == 0) | (chunk != jnp.roll(chunk, 1))
    sent = (((1 << 13) - 1) << 10) | _B_PER_W
    ckey = (widx << 23) | jnp.where(
        newf, (chunk << 10) | (pos % _B_PER_W), sent)
    csort = jnp.sort(ckey)
    dense = (csort >> 10) & ((1 << 13) - 1)
    estart = csort & ((1 << 10) - 1)
    counts = newf.reshape(_NW, _B_PER_W).sum(axis=1).astype(jnp.int32)
    counts_pad = jnp.pad(counts[:, None], ((0, 0), (0, 15)))
    return (sids, dense, estart, order.astype(jnp.int32),
            counts_pad.reshape(-1))


def kernel(u_ids, i_ids, user_table, item_table):
    pu = _prep(u_ids)
    pi = _prep(i_ids)
    u_inter, i_inter = _gather_sorted(
        *pu, *pi, user_table.T, item_table.T)
    u_t, i_t = _transpose_out(u_inter, i_inter)
    return (u_t.T, i_t.T)
